# baseline (speedup 1.0000x reference)
import jax
import jax.numpy as jnp
from jax import lax
from jax.experimental import pallas as pl
from jax.experimental.pallas import tpu as pltpu
from jax.experimental.pallas import tpu_sc as plsc

B = 64
N = 224 * 224
NB = 1008
LANES = 16
NC = 2
NS = 16
NW = NC * NS
SPW = B // NW
NVEC = N // LANES
EPS = 1e-12


def _bce_body(p_ref, t_ref, o_ref):
    p = p_ref[...]
    t = t_ref[...].astype(jnp.float32)
    bce = jax.nn.relu(p) - p * t + jnp.log1p(jnp.exp(-jnp.abs(p)))
    bce_s = jnp.sum(bce, axis=(1, 2))
    lane = lax.broadcasted_iota(jnp.int32, (8, 128), 1)
    o_ref[...] = jnp.where(lane == 0, bce_s[:, None], 0.0)


def _bce_call(pred, target):
    return pl.pallas_call(
        _bce_body,
        grid=(B // 8,),
        in_specs=[
            pl.BlockSpec((8, 224, 224), lambda i: (i, 0, 0)),
            pl.BlockSpec((8, 224, 224), lambda i: (i, 0, 0)),
        ],
        out_specs=pl.BlockSpec((8, 128), lambda i: (i, 0)),
        out_shape=jax.ShapeDtypeStruct((B, 128), jnp.float32),
    )(pred, target)


def _bcast_lane(x, lane):
    idx = jnp.full((LANES, 1), lane, dtype=jnp.int32)
    dnums = lax.GatherDimensionNumbers(
        offset_dims=(), collapsed_slice_dims=(0,), start_index_map=(0,))
    return lax.gather(x, idx, dnums, slice_sizes=(1,),
                      mode=lax.GatherScatterMode.PROMISE_IN_BOUNDS)


NCH = 4
RCH = 224 // NCH


def _sc_body(pred_hbm, tgt_hbm, out_hbm, ebuf, vbuf, hist, obuf, *sems):
    wid = lax.axis_index("s") * NC + lax.axis_index("c")
    lane_iota = lax.iota(jnp.int32, LANES)
    lane_base = lane_iota * NB
    zero_i = jnp.zeros((LANES,), jnp.int32)
    zero_f = jnp.zeros((LANES,), jnp.float32)

    pending = {}

    def issue(k, c):
        s = wid * SPW + k
        rs = pl.ds(c * RCH, RCH)
        pending[(k, c)] = (
            pltpu.async_copy(pred_hbm.at[s, rs], ebuf.at[rs], sems[c]),
            pltpu.async_copy(tgt_hbm.at[s, rs], vbuf.at[rs], sems[c]),
        )

    issue(0, 0)
    issue(0, 1)

    @plsc.parallel_loop(0, NB)
    def _zero(i):
        hist[pl.ds(i * LANES, LANES)] = zero_i

    for k in range(SPW):
        s = wid * SPW + k

        carry_a = tuple([zero_f - 1e30] * 4 + [zero_i] * 4)
        for c in range(NCH):
            cp, ct = pending.pop((k, c))
            cp.wait()
            ct.wait()
            if k == 0 and c + 2 < NCH:
                issue(0, c + 2)

            @plsc.parallel_loop(c * RCH, (c + 1) * RCH, carry=carry_a)
            def _passa(r, cr):
                accs = list(cr)
                for v in range(14):
                    p = ebuf[r, pl.ds(v * LANES, LANES)]
                    t = vbuf[r, pl.ds(v * LANES, LANES)]
                    tf = t.astype(jnp.float32)
                    e = 1.0 - p * (tf + tf - 1.0)
                    ebuf[r, pl.ds(v * LANES, LANES)] = e
                    vbuf[r, pl.ds(v * LANES, LANES)] = (t << 16) + 1
                    accs[v % 4] = jnp.maximum(accs[v % 4], e)
                    accs[4 + v % 4] = accs[4 + v % 4] + t
                return tuple(accs)

            carry_a = _passa

        maxv = jnp.maximum(jnp.maximum(carry_a[0], carry_a[1]),
                           jnp.maximum(carry_a[2], carry_a[3]))
        psum = (carry_a[4] + carry_a[5]) + (carry_a[6] + carry_a[7])
        max_e = _bcast_lane(plsc.cummax(maxv), LANES - 1)
        pos_p = _bcast_lane(jnp.cumsum(psum), LANES - 1).astype(jnp.float32)
        scale = NB / jnp.maximum(max_e, 1e-20)

        for c in range(NCH):

            @plsc.parallel_loop(c * RCH, (c + 1) * RCH)
            def _passb(r):
                for v in range(14):
                    e = ebuf[r, pl.ds(v * LANES, LANES)]
                    val = vbuf[r, pl.ds(v * LANES, LANES)]
                    msk = e > 0.0
                    bi = jnp.clip((e * scale).astype(jnp.int32), 0, NB - 1)
                    plsc.addupdate_scatter(hist, [lane_base + bi], val,
                                           mask=msk)

            if k + 1 < SPW:
                issue(k + 1, c)

        @plsc.parallel_loop(0, NB // LANES, carry=(zero_i, zero_i, zero_f))
        def _jac(cb, carry):
            ccnt, cpos, accj = carry
            bucket0 = NB - LANES - cb * LANES
            parts = [zero_i] * 4
            for l in range(LANES):
                off = l * NB + bucket0
                parts[l % 4] = parts[l % 4] + hist[pl.ds(off, LANES)]
                hist[pl.ds(off, LANES)] = zero_i
            packed = (parts[0] + parts[1]) + (parts[2] + parts[3])
            cnt = packed & 0xFFFF
            pos = lax.shift_right_logical(packed, 16)
            cs = jnp.cumsum(lax.rev(cnt, (0,)))
            ps = jnp.cumsum(lax.rev(pos, (0,)))
            ccum = cs + ccnt
            pcum = ps + cpos
            i_f = ccum.astype(jnp.float32)
            c_f = pcum.astype(jnp.float32)
            j = 1.0 - (pos_p - c_f) / jnp.maximum(pos_p + i_f - c_f, EPS)
            return (ccnt + _bcast_lane(cs, LANES - 1),
                    cpos + _bcast_lane(ps, LANES - 1),
                    accj + j)

        ccnt, cpos, accj = _jac
        sum_j = _bcast_lane(jnp.cumsum(accj), LANES - 1)
        t_f = ccnt.astype(jnp.float32)
        c_f = cpos.astype(jnp.float32)
        j_bot = 1.0 - (pos_p - c_f) / jnp.maximum(pos_p + t_f - c_f, EPS)
        delta = jnp.maximum(max_e, 1e-20) * (1.0 / NB)
        lov = delta * sum_j - 0.5 * delta * j_bot
        lov = jnp.where(t_f > 0.0, lov, 0.0)

        obuf[pl.ds(0, LANES)] = jnp.where(lane_iota == 0, lov, 0.0)
        for z in range(1, 8):
            obuf[pl.ds(z * LANES, LANES)] = zero_f
        pltpu.sync_copy(obuf, out_hbm.at[s])


def _sc_call(pred, target):
    mesh = plsc.VectorSubcoreMesh(core_axis_name="c", subcore_axis_name="s")
    kern = pl.kernel(
        _sc_body,
        out_type=jax.ShapeDtypeStruct((B, 128), jnp.float32),
        mesh=mesh,
        scratch_types=[
            pltpu.VMEM((224, 224), jnp.float32),
            pltpu.VMEM((224, 224), jnp.int32),
            pltpu.VMEM((NB * LANES,), jnp.int32),
            pltpu.VMEM((128,), jnp.float32),
            pltpu.SemaphoreType.DMA,
            pltpu.SemaphoreType.DMA,
            pltpu.SemaphoreType.DMA,
            pltpu.SemaphoreType.DMA,
        ],
        compiler_params=pltpu.CompilerParams(needs_layout_passes=False),
    )
    return kern(pred, target)


def _combine_body(bce_ref, lov_ref, o_ref):
    bce = bce_ref[...]
    lov = lov_ref[...]
    lane = lax.broadcasted_iota(jnp.int32, (B, 128), 1)
    bce_sum = jnp.sum(jnp.where(lane == 0, bce, 0.0))
    lov_sum = jnp.sum(jnp.where(lane == 0, lov, 0.0))
    o_ref[...] = jnp.full((1, 1), (bce_sum / N + lov_sum) / B, jnp.float32)


def _combine_call(bce, lov):
    return pl.pallas_call(
        _combine_body,
        out_shape=jax.ShapeDtypeStruct((1, 1), jnp.float32),
    )(bce, lov)


@jax.jit
def kernel(pred, target):
    bce = _bce_call(pred, target)
    lov = _sc_call(pred, target)
    loss = _combine_call(bce, lov)
    return loss[0, 0]
